# Initial kernel scaffold; baseline (speedup 1.0000x reference)
import jax
import jax.numpy as jnp
from jax.experimental import pallas as pl


def kernel(feat, edge_index, edge_feat, W, b):
    raise NotImplementedError("write your pallas kernel here")


# speedup vs baseline: 5.0804x; 5.0804x over previous
import functools

import jax
import jax.numpy as jnp
from jax import lax
from jax.experimental import pallas as pl
from jax.experimental.pallas import tpu as pltpu
from jax.experimental.pallas import tpu_sc as plsc

NC = 2
NS = 16
CHUNK = 80


def _sc_segment_sums(N, D, E, DE):
    NW = NC * NS
    ep_tile = E // NW
    n_chunks = ep_tile // CHUNK
    rpt = N // NS

    mesh = plsc.VectorSubcoreMesh(
        core_axis_name="c", subcore_axis_name="s", num_cores=NC, num_subcores=NS
    )

    @functools.partial(
        pl.kernel,
        out_type=(
            jax.ShapeDtypeStruct((NC, N, D), jnp.float32),
            jax.ShapeDtypeStruct((NC, N, DE), jnp.float32),
        ),
        mesh=mesh,
        compiler_params=pltpu.CompilerParams(use_tc_tiling_on_sc=False),
        scratch_types=[
            pltpu.VMEM_SHARED((N, D), jnp.float32),
            pltpu.VMEM_SHARED((N, DE), jnp.float32),
            pltpu.VMEM((n_chunks, CHUNK), jnp.int32),
            pltpu.VMEM((n_chunks, CHUNK), jnp.int32),
            pltpu.VMEM((CHUNK, D), jnp.float32),
            pltpu.VMEM((CHUNK, DE), jnp.float32),
            pltpu.SemaphoreType.DMA,
            pltpu.SemaphoreType.DMA,
        ],
    )
    def sc_kernel(feat_hbm, src_hbm, dst_hbm, ef_hbm, zf_hbm, ze_hbm,
                  hf_out, he_out, acc_f, acc_e, src_v, dst_v, rows_v, ef_v,
                  gsem, esem):
        c = lax.axis_index("c")
        s = lax.axis_index("s")
        wid = c * NS + s

        pltpu.sync_copy(zf_hbm, acc_f.at[pl.ds(s * rpt, rpt)])
        pltpu.sync_copy(ze_hbm, acc_e.at[pl.ds(s * rpt, rpt)])
        pltpu.sync_copy(src_hbm.at[wid], src_v)
        pltpu.sync_copy(dst_hbm.at[wid], dst_v)
        plsc.subcore_barrier()

        def body(j, carry):
            base = wid * ep_tile + j * CHUNK
            pltpu.async_copy(feat_hbm.at[src_v.at[j]], rows_v, gsem).wait()
            pltpu.async_copy(ef_hbm.at[pl.ds(base, CHUNK)], ef_v, esem).wait()
            pltpu.sync_copy(rows_v, acc_f.at[dst_v.at[j]], add=True)
            pltpu.sync_copy(ef_v, acc_e.at[dst_v.at[j]], add=True)
            return carry

        lax.fori_loop(0, n_chunks, body, 0)
        plsc.subcore_barrier()

        sl = pl.ds(s * rpt, rpt)
        pltpu.sync_copy(acc_f.at[sl], hf_out.at[c, sl])
        pltpu.sync_copy(acc_e.at[sl], he_out.at[c, sl])

    return sc_kernel


def _tc_combine(N, D, DE, R=1000):
    def body(feat_ref, hf_ref, he_ref, w1_ref, w2_ref, b_ref, out_ref):
        hf = hf_ref[0] + hf_ref[1]
        he = he_ref[0] + he_ref[1]
        acc = jnp.dot(hf, w1_ref[...], preferred_element_type=jnp.float32)
        acc = acc + jnp.dot(he, w2_ref[...], preferred_element_type=jnp.float32)
        out_ref[...] = feat_ref[...] + acc + b_ref[...]

    return pl.pallas_call(
        body,
        grid=(N // R,),
        in_specs=[
            pl.BlockSpec((R, D), lambda i: (i, 0)),
            pl.BlockSpec((NC, R, D), lambda i: (0, i, 0)),
            pl.BlockSpec((NC, R, DE), lambda i: (0, i, 0)),
            pl.BlockSpec((D, D), lambda i: (0, 0)),
            pl.BlockSpec((DE, D), lambda i: (0, 0)),
            pl.BlockSpec((1, D), lambda i: (0, 0)),
        ],
        out_specs=pl.BlockSpec((R, D), lambda i: (i, 0)),
        out_shape=jax.ShapeDtypeStruct((N, D), jnp.float32),
    )


def kernel(feat, edge_index, edge_feat, W, b):
    N, D = feat.shape
    E, DE = edge_feat.shape
    NW = NC * NS
    npad = -(-N // (8 * NS)) * (8 * NS)

    nch = E // (NW * CHUNK)
    src = edge_index[0].astype(jnp.int32).reshape(NW, nch, CHUNK)
    dst = edge_index[1].astype(jnp.int32).reshape(NW, nch, CHUNK)
    zeros_f = jnp.zeros((npad // NS, D), jnp.float32)
    zeros_e = jnp.zeros((npad // NS, DE), jnp.float32)

    hf, he = _sc_segment_sums(npad, D, E, DE)(
        feat, src, dst, edge_feat, zeros_f, zeros_e
    )
    return _tc_combine(N, D, DE)(
        feat, hf, he, W[:D], W[D:], b.reshape(1, D)
    )
